# initial kernel scaffold (unmeasured)
import jax
import jax.numpy as jnp
from jax import lax
from jax.experimental import pallas as pl
from jax.experimental.pallas import tpu as pltpu


def kernel(
    x,
):
    def body(*refs):
        pass

    out_shape = jax.ShapeDtypeStruct(..., jnp.float32)
    return pl.pallas_call(body, out_shape=out_shape)(...)


# baseline (device time: 605681 ns/iter reference)
import jax
import jax.numpy as jnp
from jax import lax
from jax.experimental import pallas as pl
from jax.experimental.pallas import tpu as pltpu

Z = 4


def kernel(x):
    m_per, n = x.shape

    def body(x_ref, out_ref, copy_sem, send_sems, recv_sems):
        my_x = lax.axis_index("x")
        my_y = lax.axis_index("y")
        my_z = lax.axis_index("z")
        left = (my_z - 1) % Z
        right = (my_z + 1) % Z

        barrier_sem = pltpu.get_barrier_semaphore()
        for nbr in (left, right):
            pl.semaphore_signal(
                barrier_sem, inc=1,
                device_id=(my_x, my_y, nbr),
                device_id_type=pl.DeviceIdType.MESH,
            )
        pl.semaphore_wait(barrier_sem, 2)

        local = pltpu.make_async_copy(
            x_ref, out_ref.at[pl.ds(my_z * m_per, m_per), :], copy_sem
        )
        local.start()
        local.wait()

        for h in range(Z - 1):
            src_origin = (my_z - h) % Z
            rdma = pltpu.make_async_remote_copy(
                src_ref=out_ref.at[pl.ds(src_origin * m_per, m_per), :],
                dst_ref=out_ref.at[pl.ds(src_origin * m_per, m_per), :],
                send_sem=send_sems.at[h],
                recv_sem=recv_sems.at[h],
                device_id=(my_x, my_y, right),
                device_id_type=pl.DeviceIdType.MESH,
            )
            rdma.start()
            rdma.wait()

    return pl.pallas_call(
        body,
        out_shape=jax.ShapeDtypeStruct((Z * m_per, n), x.dtype),
        in_specs=[pl.BlockSpec(memory_space=pltpu.MemorySpace.HBM)],
        out_specs=pl.BlockSpec(memory_space=pltpu.MemorySpace.HBM),
        scratch_shapes=[
            pltpu.SemaphoreType.DMA,
            pltpu.SemaphoreType.DMA((Z - 1,)),
            pltpu.SemaphoreType.DMA((Z - 1,)),
        ],
        compiler_params=pltpu.CompilerParams(collective_id=0),
    )(x)


# device time: 421989 ns/iter; 1.4353x vs baseline; 1.4353x over previous
import jax
import jax.numpy as jnp
from jax import lax
from jax.experimental import pallas as pl
from jax.experimental.pallas import tpu as pltpu

Z = 4
Q = 1


def kernel(x):
    m_per, n = x.shape
    half = m_per // 2
    rp = half // Q

    def body(x_ref, out_ref, copy_sem, rsend, rrecv, lsend, lrecv, xsend, xrecv):
        my_x = lax.axis_index("x")
        my_y = lax.axis_index("y")
        my_z = lax.axis_index("z")
        h = my_x
        left = (my_z - 1) % Z
        right = (my_z + 1) % Z
        xn = 1 - my_x

        local = pltpu.make_async_copy(
            x_ref, out_ref.at[pl.ds(my_z * m_per, m_per), :], copy_sem
        )
        local.start()

        barrier_sem = pltpu.get_barrier_semaphore()
        for nbr in ((my_x, my_y, left), (my_x, my_y, right), (xn, my_y, my_z)):
            pl.semaphore_signal(
                barrier_sem, inc=1, device_id=nbr,
                device_id_type=pl.DeviceIdType.MESH,
            )
        pl.semaphore_wait(barrier_sem, 3)

        def piece_out(c, k, hh):
            return out_ref.at[pl.ds(c * m_per + hh * half + k * rp, rp), :]

        def piece_own(k):
            return x_ref.at[pl.ds(h * half + k * rp, rp), :]

        def rdma(src, dst, ssem, rsem, dev):
            return pltpu.make_async_remote_copy(
                src_ref=src, dst_ref=dst, send_sem=ssem, recv_sem=rsem,
                device_id=dev, device_id_type=pl.DeviceIdType.MESH,
            )

        z_right = (my_x, my_y, my_z + 1)
        z_left = (my_x, my_y, my_z - 1)
        x_dev = (xn, my_y, my_z)

        for k in range(Q):
            @pl.when(my_z < Z - 1)
            def _():
                rdma(piece_own(k), piece_out(my_z, k, h),
                     rsend.at[k], rrecv.at[k], z_right).start()

            @pl.when(my_z > 0)
            def _():
                rdma(piece_own(k), piece_out(my_z, k, h),
                     lsend.at[k], lrecv.at[k], z_left).start()

        for t in range(1, Z):
            for k in range(Q):
                @pl.when(t <= my_z)
                def _():
                    c = my_z - t
                    dst = piece_out(c, k, h)
                    rdma(dst, dst, rsend.at[0], rrecv.at[(t - 1) * Q + k],
                         z_left).wait_recv()
                    rdma(dst, piece_out(c, k, h),
                         xsend.at[(t - 1) * Q + k],
                         xrecv.at[(t - 1) * Q + k], x_dev).start()

                @pl.when(jnp.logical_and(t <= my_z, my_z < Z - 1))
                def _():
                    c = my_z - t
                    src = piece_out(c, k, h)
                    rdma(src, src, rsend.at[t * Q + k],
                         rrecv.at[t * Q + k], z_right).start()

                @pl.when(my_z + t <= Z - 1)
                def _():
                    c = my_z + t
                    dst = piece_out(c, k, h)
                    rdma(dst, dst, lsend.at[0], lrecv.at[(t - 1) * Q + k],
                         z_right).wait_recv()
                    rdma(dst, piece_out(c, k, h),
                         xsend.at[3 * Q + (t - 1) * Q + k],
                         xrecv.at[3 * Q + (t - 1) * Q + k], x_dev).start()

                @pl.when(jnp.logical_and(my_z + t <= Z - 1, my_z > 0))
                def _():
                    c = my_z + t
                    src = piece_out(c, k, h)
                    rdma(src, src, lsend.at[t * Q + k],
                         lrecv.at[t * Q + k], z_left).start()

        for t in range(1, Z):
            for k in range(Q):
                @pl.when(t <= my_z)
                def _():
                    c = my_z - t
                    dst = piece_out(c, k, 1 - h)
                    rdma(dst, dst, xsend.at[0],
                         xrecv.at[(t - 1) * Q + k], x_dev).wait_recv()

                @pl.when(my_z + t <= Z - 1)
                def _():
                    c = my_z + t
                    dst = piece_out(c, k, 1 - h)
                    rdma(dst, dst, xsend.at[0],
                         xrecv.at[3 * Q + (t - 1) * Q + k], x_dev).wait_recv()

        for k in range(Q):
            @pl.when(my_z < Z - 1)
            def _():
                rdma(piece_own(k), piece_own(k), rsend.at[k],
                     rrecv.at[0], z_right).wait_send()

            @pl.when(my_z > 0)
            def _():
                rdma(piece_own(k), piece_own(k), lsend.at[k],
                     lrecv.at[0], z_left).wait_send()

        for t in range(1, Z):
            for k in range(Q):
                @pl.when(jnp.logical_and(t <= my_z, my_z < Z - 1))
                def _():
                    src = piece_out(my_z - t, k, h)
                    rdma(src, src, rsend.at[t * Q + k], rrecv.at[0],
                         z_right).wait_send()

                @pl.when(t <= my_z)
                def _():
                    src = piece_out(my_z - t, k, h)
                    rdma(src, src, xsend.at[(t - 1) * Q + k], xrecv.at[0],
                         x_dev).wait_send()

                @pl.when(jnp.logical_and(my_z + t <= Z - 1, my_z > 0))
                def _():
                    src = piece_out(my_z + t, k, h)
                    rdma(src, src, lsend.at[t * Q + k], lrecv.at[0],
                         z_left).wait_send()

                @pl.when(my_z + t <= Z - 1)
                def _():
                    src = piece_out(my_z + t, k, h)
                    rdma(src, src, xsend.at[3 * Q + (t - 1) * Q + k],
                         xrecv.at[0], x_dev).wait_send()

        local.wait()

    nsl = 3 * Q
    return pl.pallas_call(
        body,
        out_shape=jax.ShapeDtypeStruct((Z * m_per, n), x.dtype),
        in_specs=[pl.BlockSpec(memory_space=pltpu.MemorySpace.HBM)],
        out_specs=pl.BlockSpec(memory_space=pltpu.MemorySpace.HBM),
        scratch_shapes=[
            pltpu.SemaphoreType.DMA,
            pltpu.SemaphoreType.DMA((nsl,)),
            pltpu.SemaphoreType.DMA((nsl,)),
            pltpu.SemaphoreType.DMA((nsl,)),
            pltpu.SemaphoreType.DMA((nsl,)),
            pltpu.SemaphoreType.DMA((2 * nsl,)),
            pltpu.SemaphoreType.DMA((2 * nsl,)),
        ],
        compiler_params=pltpu.CompilerParams(collective_id=0),
    )(x)


# device time: 351066 ns/iter; 1.7253x vs baseline; 1.2020x over previous
import jax
import jax.numpy as jnp
from jax import lax
from jax.experimental import pallas as pl
from jax.experimental.pallas import tpu as pltpu

Z = 4
Q = 4


def kernel(x):
    m_per, n = x.shape
    half = m_per // 2
    rp = half // Q

    def body(x_ref, out_ref, copy_sem, rsend, rrecv, lsend, lrecv, xsend, xrecv):
        my_x = lax.axis_index("x")
        my_y = lax.axis_index("y")
        my_z = lax.axis_index("z")
        h = my_x
        left = (my_z - 1) % Z
        right = (my_z + 1) % Z
        xn = 1 - my_x

        local = pltpu.make_async_copy(
            x_ref, out_ref.at[pl.ds(my_z * m_per, m_per), :], copy_sem
        )
        local.start()

        barrier_sem = pltpu.get_barrier_semaphore()
        for nbr in ((my_x, my_y, left), (my_x, my_y, right), (xn, my_y, my_z)):
            pl.semaphore_signal(
                barrier_sem, inc=1, device_id=nbr,
                device_id_type=pl.DeviceIdType.MESH,
            )
        pl.semaphore_wait(barrier_sem, 3)

        def piece_out(c, k, hh):
            return out_ref.at[pl.ds(c * m_per + hh * half + k * rp, rp), :]

        def piece_own(k):
            return x_ref.at[pl.ds(h * half + k * rp, rp), :]

        def rdma(src, dst, ssem, rsem, dev):
            return pltpu.make_async_remote_copy(
                src_ref=src, dst_ref=dst, send_sem=ssem, recv_sem=rsem,
                device_id=dev, device_id_type=pl.DeviceIdType.MESH,
            )

        z_right = (my_x, my_y, my_z + 1)
        z_left = (my_x, my_y, my_z - 1)
        x_dev = (xn, my_y, my_z)

        for k in range(Q):
            @pl.when(my_z < Z - 1)
            def _():
                rdma(piece_own(k), piece_out(my_z, k, h),
                     rsend.at[k], rrecv.at[k], z_right).start()

            @pl.when(my_z > 0)
            def _():
                rdma(piece_own(k), piece_out(my_z, k, h),
                     lsend.at[k], lrecv.at[k], z_left).start()

        for t in range(1, Z):
            for k in range(Q):
                @pl.when(t <= my_z)
                def _():
                    c = my_z - t
                    dst = piece_out(c, k, h)
                    rdma(dst, dst, rsend.at[0], rrecv.at[(t - 1) * Q + k],
                         z_left).wait_recv()
                    rdma(dst, piece_out(c, k, h),
                         xsend.at[(t - 1) * Q + k],
                         xrecv.at[(t - 1) * Q + k], x_dev).start()

                @pl.when(jnp.logical_and(t <= my_z, my_z < Z - 1))
                def _():
                    c = my_z - t
                    src = piece_out(c, k, h)
                    rdma(src, src, rsend.at[t * Q + k],
                         rrecv.at[t * Q + k], z_right).start()

                @pl.when(my_z + t <= Z - 1)
                def _():
                    c = my_z + t
                    dst = piece_out(c, k, h)
                    rdma(dst, dst, lsend.at[0], lrecv.at[(t - 1) * Q + k],
                         z_right).wait_recv()
                    rdma(dst, piece_out(c, k, h),
                         xsend.at[3 * Q + (t - 1) * Q + k],
                         xrecv.at[3 * Q + (t - 1) * Q + k], x_dev).start()

                @pl.when(jnp.logical_and(my_z + t <= Z - 1, my_z > 0))
                def _():
                    c = my_z + t
                    src = piece_out(c, k, h)
                    rdma(src, src, lsend.at[t * Q + k],
                         lrecv.at[t * Q + k], z_left).start()

        for t in range(1, Z):
            for k in range(Q):
                @pl.when(t <= my_z)
                def _():
                    c = my_z - t
                    dst = piece_out(c, k, 1 - h)
                    rdma(dst, dst, xsend.at[0],
                         xrecv.at[(t - 1) * Q + k], x_dev).wait_recv()

                @pl.when(my_z + t <= Z - 1)
                def _():
                    c = my_z + t
                    dst = piece_out(c, k, 1 - h)
                    rdma(dst, dst, xsend.at[0],
                         xrecv.at[3 * Q + (t - 1) * Q + k], x_dev).wait_recv()

        for k in range(Q):
            @pl.when(my_z < Z - 1)
            def _():
                rdma(piece_own(k), piece_own(k), rsend.at[k],
                     rrecv.at[0], z_right).wait_send()

            @pl.when(my_z > 0)
            def _():
                rdma(piece_own(k), piece_own(k), lsend.at[k],
                     lrecv.at[0], z_left).wait_send()

        for t in range(1, Z):
            for k in range(Q):
                @pl.when(jnp.logical_and(t <= my_z, my_z < Z - 1))
                def _():
                    src = piece_out(my_z - t, k, h)
                    rdma(src, src, rsend.at[t * Q + k], rrecv.at[0],
                         z_right).wait_send()

                @pl.when(t <= my_z)
                def _():
                    src = piece_out(my_z - t, k, h)
                    rdma(src, src, xsend.at[(t - 1) * Q + k], xrecv.at[0],
                         x_dev).wait_send()

                @pl.when(jnp.logical_and(my_z + t <= Z - 1, my_z > 0))
                def _():
                    src = piece_out(my_z + t, k, h)
                    rdma(src, src, lsend.at[t * Q + k], lrecv.at[0],
                         z_left).wait_send()

                @pl.when(my_z + t <= Z - 1)
                def _():
                    src = piece_out(my_z + t, k, h)
                    rdma(src, src, xsend.at[3 * Q + (t - 1) * Q + k],
                         xrecv.at[0], x_dev).wait_send()

        local.wait()

    nsl = 3 * Q
    return pl.pallas_call(
        body,
        out_shape=jax.ShapeDtypeStruct((Z * m_per, n), x.dtype),
        in_specs=[pl.BlockSpec(memory_space=pltpu.MemorySpace.HBM)],
        out_specs=pl.BlockSpec(memory_space=pltpu.MemorySpace.HBM),
        scratch_shapes=[
            pltpu.SemaphoreType.DMA,
            pltpu.SemaphoreType.DMA((nsl,)),
            pltpu.SemaphoreType.DMA((nsl,)),
            pltpu.SemaphoreType.DMA((nsl,)),
            pltpu.SemaphoreType.DMA((nsl,)),
            pltpu.SemaphoreType.DMA((2 * nsl,)),
            pltpu.SemaphoreType.DMA((2 * nsl,)),
        ],
        compiler_params=pltpu.CompilerParams(collective_id=0),
    )(x)


# device time: 311051 ns/iter; 1.9472x vs baseline; 1.1286x over previous
import jax
import jax.numpy as jnp
from jax import lax
from jax.experimental import pallas as pl
from jax.experimental.pallas import tpu as pltpu

Z = 4
Q = 1


def kernel(x):
    m_per, n = x.shape
    qh = m_per // 4
    rp = qh // Q
    hp = rp // 2

    def body(x_ref, out_ref, copy_sem,
             rsend, rrecv, lsend, lrecv,
             xdsend, xdrecv, ydsend, ydrecv,
             xhsend, xhrecv, yhsend, yhrecv):
        my_x = lax.axis_index("x")
        my_y = lax.axis_index("y")
        my_z = lax.axis_index("z")
        qi = 2 * my_x + my_y
        q_x = 2 * (1 - my_x) + my_y
        q_y = 2 * my_x + (1 - my_y)
        q_d = 2 * (1 - my_x) + (1 - my_y)
        left = (my_z - 1) % Z
        right = (my_z + 1) % Z

        z_right = (my_x, my_y, my_z + 1)
        z_left = (my_x, my_y, my_z - 1)
        x_dev = (1 - my_x, my_y, my_z)
        y_dev = (my_x, 1 - my_y, my_z)

        local = pltpu.make_async_copy(
            x_ref, out_ref.at[pl.ds(my_z * m_per, m_per), :], copy_sem
        )
        local.start()

        barrier_sem = pltpu.get_barrier_semaphore()
        for nbr in ((my_x, my_y, left), (my_x, my_y, right), x_dev, y_dev):
            pl.semaphore_signal(
                barrier_sem, inc=1, device_id=nbr,
                device_id_type=pl.DeviceIdType.MESH,
            )
        pl.semaphore_wait(barrier_sem, 4)

        def qslice(c, q, k):
            return out_ref.at[pl.ds(c * m_per + q * qh + k * rp, rp), :]

        def hslice(c, q, k, hh):
            return out_ref.at[pl.ds(c * m_per + q * qh + k * rp + hh * hp, hp), :]

        def own(k):
            return x_ref.at[pl.ds(qi * qh + k * rp, rp), :]

        def rdma(src, dst, ssem, rsem, dev):
            return pltpu.make_async_remote_copy(
                src_ref=src, dst_ref=dst, send_sem=ssem, recv_sem=rsem,
                device_id=dev, device_id_type=pl.DeviceIdType.MESH,
            )

        for k in range(Q):
            @pl.when(my_z < Z - 1)
            def _():
                rdma(own(k), qslice(my_z, qi, k),
                     rsend.at[k], rrecv.at[k], z_right).start()

            @pl.when(my_z > 0)
            def _():
                rdma(own(k), qslice(my_z, qi, k),
                     lsend.at[k], lrecv.at[k], z_left).start()

        for t in range(1, Z):
            for k in range(Q):
                s = (t - 1) * Q + k

                @pl.when(t <= my_z)
                def _():
                    c = my_z - t
                    dst = qslice(c, qi, k)
                    rdma(dst, dst, rsend.at[0], rrecv.at[s], z_left).wait_recv()
                    rdma(dst, qslice(c, qi, k),
                         xdsend.at[s], xdrecv.at[s], x_dev).start()
                    rdma(dst, qslice(c, qi, k),
                         ydsend.at[s], ydrecv.at[s], y_dev).start()

                @pl.when(jnp.logical_and(t <= my_z, my_z < Z - 1))
                def _():
                    src = qslice(my_z - t, qi, k)
                    rdma(src, src, rsend.at[t * Q + k],
                         rrecv.at[t * Q + k], z_right).start()

                @pl.when(my_z + t <= Z - 1)
                def _():
                    c = my_z + t
                    dst = qslice(c, qi, k)
                    rdma(dst, dst, lsend.at[0], lrecv.at[s], z_right).wait_recv()
                    rdma(dst, qslice(c, qi, k),
                         xdsend.at[3 * Q + s], xdrecv.at[3 * Q + s],
                         x_dev).start()
                    rdma(dst, qslice(c, qi, k),
                         ydsend.at[3 * Q + s], ydrecv.at[3 * Q + s],
                         y_dev).start()

                @pl.when(jnp.logical_and(my_z + t <= Z - 1, my_z > 0))
                def _():
                    src = qslice(my_z + t, qi, k)
                    rdma(src, src, lsend.at[t * Q + k],
                         lrecv.at[t * Q + k], z_left).start()

        for t in range(1, Z):
            for k in range(Q):
                s = (t - 1) * Q + k
                for ss, cc, rightward in (
                    (s, my_z - t, True), (3 * Q + s, my_z + t, False)
                ):
                    guard = (t <= my_z) if rightward else (my_z + t <= Z - 1)

                    @pl.when(guard)
                    def _(ss=ss, cc=cc):
                        dx = qslice(cc, q_x, k)
                        rdma(dx, dx, xdsend.at[0], xdrecv.at[ss],
                             x_dev).wait_recv()
                        hsrc = hslice(cc, q_x, k, 0)
                        rdma(hsrc, hslice(cc, q_x, k, 0),
                             yhsend.at[ss], yhrecv.at[ss], y_dev).start()

                        dy = qslice(cc, q_y, k)
                        rdma(dy, dy, ydsend.at[0], ydrecv.at[ss],
                             y_dev).wait_recv()
                        hsrc2 = hslice(cc, q_y, k, 1)
                        rdma(hsrc2, hslice(cc, q_y, k, 1),
                             xhsend.at[ss], xhrecv.at[ss], x_dev).start()

        for t in range(1, Z):
            for k in range(Q):
                s = (t - 1) * Q + k
                for ss, cc, rightward in (
                    (s, my_z - t, True), (3 * Q + s, my_z + t, False)
                ):
                    guard = (t <= my_z) if rightward else (my_z + t <= Z - 1)

                    @pl.when(guard)
                    def _(ss=ss, cc=cc):
                        d0 = hslice(cc, q_d, k, 0)
                        rdma(d0, d0, yhsend.at[0], yhrecv.at[ss],
                             y_dev).wait_recv()
                        d1 = hslice(cc, q_d, k, 1)
                        rdma(d1, d1, xhsend.at[0], xhrecv.at[ss],
                             x_dev).wait_recv()

        for k in range(Q):
            @pl.when(my_z < Z - 1)
            def _():
                rdma(own(k), own(k), rsend.at[k], rrecv.at[0],
                     z_right).wait_send()

            @pl.when(my_z > 0)
            def _():
                rdma(own(k), own(k), lsend.at[k], lrecv.at[0],
                     z_left).wait_send()

        for t in range(1, Z):
            for k in range(Q):
                s = (t - 1) * Q + k

                @pl.when(jnp.logical_and(t <= my_z, my_z < Z - 1))
                def _():
                    src = qslice(my_z - t, qi, k)
                    rdma(src, src, rsend.at[t * Q + k], rrecv.at[0],
                         z_right).wait_send()

                @pl.when(jnp.logical_and(my_z + t <= Z - 1, my_z > 0))
                def _():
                    src = qslice(my_z + t, qi, k)
                    rdma(src, src, lsend.at[t * Q + k], lrecv.at[0],
                         z_left).wait_send()

                for ss, cc, rightward in (
                    (s, my_z - t, True), (3 * Q + s, my_z + t, False)
                ):
                    guard = (t <= my_z) if rightward else (my_z + t <= Z - 1)

                    @pl.when(guard)
                    def _(ss=ss, cc=cc):
                        src = qslice(cc, qi, k)
                        rdma(src, src, xdsend.at[ss], xdrecv.at[0],
                             x_dev).wait_send()
                        rdma(src, src, ydsend.at[ss], ydrecv.at[0],
                             y_dev).wait_send()
                        h0 = hslice(cc, q_x, k, 0)
                        rdma(h0, h0, yhsend.at[ss], yhrecv.at[0],
                             y_dev).wait_send()
                        h1 = hslice(cc, q_y, k, 1)
                        rdma(h1, h1, xhsend.at[ss], xhrecv.at[0],
                             x_dev).wait_send()

        local.wait()

    nsl = 3 * Q
    nxy = 6 * Q
    return pl.pallas_call(
        body,
        out_shape=jax.ShapeDtypeStruct((Z * m_per, n), x.dtype),
        in_specs=[pl.BlockSpec(memory_space=pltpu.MemorySpace.HBM)],
        out_specs=pl.BlockSpec(memory_space=pltpu.MemorySpace.HBM),
        scratch_shapes=[
            pltpu.SemaphoreType.DMA,
            pltpu.SemaphoreType.DMA((nsl,)),
            pltpu.SemaphoreType.DMA((nsl,)),
            pltpu.SemaphoreType.DMA((nsl,)),
            pltpu.SemaphoreType.DMA((nsl,)),
            pltpu.SemaphoreType.DMA((nxy,)),
            pltpu.SemaphoreType.DMA((nxy,)),
            pltpu.SemaphoreType.DMA((nxy,)),
            pltpu.SemaphoreType.DMA((nxy,)),
            pltpu.SemaphoreType.DMA((nxy,)),
            pltpu.SemaphoreType.DMA((nxy,)),
            pltpu.SemaphoreType.DMA((nxy,)),
            pltpu.SemaphoreType.DMA((nxy,)),
        ],
        compiler_params=pltpu.CompilerParams(collective_id=0),
    )(x)


# device time: 284296 ns/iter; 2.1305x vs baseline; 1.0941x over previous
import jax
import jax.numpy as jnp
from jax import lax
from jax.experimental import pallas as pl
from jax.experimental.pallas import tpu as pltpu

Z = 4
Q = 2


def kernel(x):
    m_per, n = x.shape
    qh = m_per // 4
    rp = qh // Q
    hp = rp // 2

    def body(x_ref, out_ref, copy_sem,
             rsend, rrecv, lsend, lrecv,
             xdsend, xdrecv, ydsend, ydrecv,
             xhsend, xhrecv, yhsend, yhrecv):
        my_x = lax.axis_index("x")
        my_y = lax.axis_index("y")
        my_z = lax.axis_index("z")
        qi = 2 * my_x + my_y
        q_x = 2 * (1 - my_x) + my_y
        q_y = 2 * my_x + (1 - my_y)
        q_d = 2 * (1 - my_x) + (1 - my_y)
        left = (my_z - 1) % Z
        right = (my_z + 1) % Z

        z_right = (my_x, my_y, my_z + 1)
        z_left = (my_x, my_y, my_z - 1)
        x_dev = (1 - my_x, my_y, my_z)
        y_dev = (my_x, 1 - my_y, my_z)

        local = pltpu.make_async_copy(
            x_ref, out_ref.at[pl.ds(my_z * m_per, m_per), :], copy_sem
        )
        local.start()

        barrier_sem = pltpu.get_barrier_semaphore()
        for nbr in ((my_x, my_y, left), (my_x, my_y, right), x_dev, y_dev):
            pl.semaphore_signal(
                barrier_sem, inc=1, device_id=nbr,
                device_id_type=pl.DeviceIdType.MESH,
            )
        pl.semaphore_wait(barrier_sem, 4)

        def qslice(c, q, k):
            return out_ref.at[pl.ds(c * m_per + q * qh + k * rp, rp), :]

        def hslice(c, q, k, hh):
            return out_ref.at[pl.ds(c * m_per + q * qh + k * rp + hh * hp, hp), :]

        def own(k):
            return x_ref.at[pl.ds(qi * qh + k * rp, rp), :]

        def rdma(src, dst, ssem, rsem, dev):
            return pltpu.make_async_remote_copy(
                src_ref=src, dst_ref=dst, send_sem=ssem, recv_sem=rsem,
                device_id=dev, device_id_type=pl.DeviceIdType.MESH,
            )

        for k in range(Q):
            @pl.when(my_z < Z - 1)
            def _():
                rdma(own(k), qslice(my_z, qi, k),
                     rsend.at[k], rrecv.at[k], z_right).start()

            @pl.when(my_z > 0)
            def _():
                rdma(own(k), qslice(my_z, qi, k),
                     lsend.at[k], lrecv.at[k], z_left).start()

        for t in range(1, Z):
            for k in range(Q):
                s = (t - 1) * Q + k

                @pl.when(t <= my_z)
                def _():
                    c = my_z - t
                    dst = qslice(c, qi, k)
                    rdma(dst, dst, rsend.at[0], rrecv.at[s], z_left).wait_recv()
                    rdma(dst, qslice(c, qi, k),
                         xdsend.at[s], xdrecv.at[s], x_dev).start()
                    rdma(dst, qslice(c, qi, k),
                         ydsend.at[s], ydrecv.at[s], y_dev).start()

                @pl.when(jnp.logical_and(t <= my_z, my_z < Z - 1))
                def _():
                    src = qslice(my_z - t, qi, k)
                    rdma(src, src, rsend.at[t * Q + k],
                         rrecv.at[t * Q + k], z_right).start()

                @pl.when(my_z + t <= Z - 1)
                def _():
                    c = my_z + t
                    dst = qslice(c, qi, k)
                    rdma(dst, dst, lsend.at[0], lrecv.at[s], z_right).wait_recv()
                    rdma(dst, qslice(c, qi, k),
                         xdsend.at[3 * Q + s], xdrecv.at[3 * Q + s],
                         x_dev).start()
                    rdma(dst, qslice(c, qi, k),
                         ydsend.at[3 * Q + s], ydrecv.at[3 * Q + s],
                         y_dev).start()

                @pl.when(jnp.logical_and(my_z + t <= Z - 1, my_z > 0))
                def _():
                    src = qslice(my_z + t, qi, k)
                    rdma(src, src, lsend.at[t * Q + k],
                         lrecv.at[t * Q + k], z_left).start()

        for t in range(1, Z):
            for k in range(Q):
                s = (t - 1) * Q + k
                for ss, cc, rightward in (
                    (s, my_z - t, True), (3 * Q + s, my_z + t, False)
                ):
                    guard = (t <= my_z) if rightward else (my_z + t <= Z - 1)

                    @pl.when(guard)
                    def _(ss=ss, cc=cc):
                        dx = qslice(cc, q_x, k)
                        rdma(dx, dx, xdsend.at[0], xdrecv.at[ss],
                             x_dev).wait_recv()
                        hsrc = hslice(cc, q_x, k, 0)
                        rdma(hsrc, hslice(cc, q_x, k, 0),
                             yhsend.at[ss], yhrecv.at[ss], y_dev).start()

                        dy = qslice(cc, q_y, k)
                        rdma(dy, dy, ydsend.at[0], ydrecv.at[ss],
                             y_dev).wait_recv()
                        hsrc2 = hslice(cc, q_y, k, 1)
                        rdma(hsrc2, hslice(cc, q_y, k, 1),
                             xhsend.at[ss], xhrecv.at[ss], x_dev).start()

        for t in range(1, Z):
            for k in range(Q):
                s = (t - 1) * Q + k
                for ss, cc, rightward in (
                    (s, my_z - t, True), (3 * Q + s, my_z + t, False)
                ):
                    guard = (t <= my_z) if rightward else (my_z + t <= Z - 1)

                    @pl.when(guard)
                    def _(ss=ss, cc=cc):
                        d0 = hslice(cc, q_d, k, 0)
                        rdma(d0, d0, yhsend.at[0], yhrecv.at[ss],
                             y_dev).wait_recv()
                        d1 = hslice(cc, q_d, k, 1)
                        rdma(d1, d1, xhsend.at[0], xhrecv.at[ss],
                             x_dev).wait_recv()

        for k in range(Q):
            @pl.when(my_z < Z - 1)
            def _():
                rdma(own(k), own(k), rsend.at[k], rrecv.at[0],
                     z_right).wait_send()

            @pl.when(my_z > 0)
            def _():
                rdma(own(k), own(k), lsend.at[k], lrecv.at[0],
                     z_left).wait_send()

        for t in range(1, Z):
            for k in range(Q):
                s = (t - 1) * Q + k

                @pl.when(jnp.logical_and(t <= my_z, my_z < Z - 1))
                def _():
                    src = qslice(my_z - t, qi, k)
                    rdma(src, src, rsend.at[t * Q + k], rrecv.at[0],
                         z_right).wait_send()

                @pl.when(jnp.logical_and(my_z + t <= Z - 1, my_z > 0))
                def _():
                    src = qslice(my_z + t, qi, k)
                    rdma(src, src, lsend.at[t * Q + k], lrecv.at[0],
                         z_left).wait_send()

                for ss, cc, rightward in (
                    (s, my_z - t, True), (3 * Q + s, my_z + t, False)
                ):
                    guard = (t <= my_z) if rightward else (my_z + t <= Z - 1)

                    @pl.when(guard)
                    def _(ss=ss, cc=cc):
                        src = qslice(cc, qi, k)
                        rdma(src, src, xdsend.at[ss], xdrecv.at[0],
                             x_dev).wait_send()
                        rdma(src, src, ydsend.at[ss], ydrecv.at[0],
                             y_dev).wait_send()
                        h0 = hslice(cc, q_x, k, 0)
                        rdma(h0, h0, yhsend.at[ss], yhrecv.at[0],
                             y_dev).wait_send()
                        h1 = hslice(cc, q_y, k, 1)
                        rdma(h1, h1, xhsend.at[ss], xhrecv.at[0],
                             x_dev).wait_send()

        local.wait()

    nsl = 3 * Q
    nxy = 6 * Q
    return pl.pallas_call(
        body,
        out_shape=jax.ShapeDtypeStruct((Z * m_per, n), x.dtype),
        in_specs=[pl.BlockSpec(memory_space=pltpu.MemorySpace.HBM)],
        out_specs=pl.BlockSpec(memory_space=pltpu.MemorySpace.HBM),
        scratch_shapes=[
            pltpu.SemaphoreType.DMA,
            pltpu.SemaphoreType.DMA((nsl,)),
            pltpu.SemaphoreType.DMA((nsl,)),
            pltpu.SemaphoreType.DMA((nsl,)),
            pltpu.SemaphoreType.DMA((nsl,)),
            pltpu.SemaphoreType.DMA((nxy,)),
            pltpu.SemaphoreType.DMA((nxy,)),
            pltpu.SemaphoreType.DMA((nxy,)),
            pltpu.SemaphoreType.DMA((nxy,)),
            pltpu.SemaphoreType.DMA((nxy,)),
            pltpu.SemaphoreType.DMA((nxy,)),
            pltpu.SemaphoreType.DMA((nxy,)),
            pltpu.SemaphoreType.DMA((nxy,)),
        ],
        compiler_params=pltpu.CompilerParams(collective_id=0),
    )(x)


# device time: 274229 ns/iter; 2.2087x vs baseline; 1.0367x over previous
import jax
import jax.numpy as jnp
from jax import lax
from jax.experimental import pallas as pl
from jax.experimental.pallas import tpu as pltpu

Z = 4
Q = 4


def kernel(x):
    m_per, n = x.shape
    qh = m_per // 4
    rp = qh // Q
    hp = rp // 2

    def body(x_ref, out_ref, copy_sem,
             rsend, rrecv, lsend, lrecv,
             xdsend, xdrecv, ydsend, ydrecv,
             xhsend, xhrecv, yhsend, yhrecv):
        my_x = lax.axis_index("x")
        my_y = lax.axis_index("y")
        my_z = lax.axis_index("z")
        qi = 2 * my_x + my_y
        q_x = 2 * (1 - my_x) + my_y
        q_y = 2 * my_x + (1 - my_y)
        q_d = 2 * (1 - my_x) + (1 - my_y)
        left = (my_z - 1) % Z
        right = (my_z + 1) % Z

        z_right = (my_x, my_y, my_z + 1)
        z_left = (my_x, my_y, my_z - 1)
        x_dev = (1 - my_x, my_y, my_z)
        y_dev = (my_x, 1 - my_y, my_z)

        local = pltpu.make_async_copy(
            x_ref, out_ref.at[pl.ds(my_z * m_per, m_per), :], copy_sem
        )
        local.start()

        barrier_sem = pltpu.get_barrier_semaphore()
        for nbr in ((my_x, my_y, left), (my_x, my_y, right), x_dev, y_dev):
            pl.semaphore_signal(
                barrier_sem, inc=1, device_id=nbr,
                device_id_type=pl.DeviceIdType.MESH,
            )
        pl.semaphore_wait(barrier_sem, 4)

        def qslice(c, q, k):
            return out_ref.at[pl.ds(c * m_per + q * qh + k * rp, rp), :]

        def hslice(c, q, k, hh):
            return out_ref.at[pl.ds(c * m_per + q * qh + k * rp + hh * hp, hp), :]

        def own(k):
            return x_ref.at[pl.ds(qi * qh + k * rp, rp), :]

        def rdma(src, dst, ssem, rsem, dev):
            return pltpu.make_async_remote_copy(
                src_ref=src, dst_ref=dst, send_sem=ssem, recv_sem=rsem,
                device_id=dev, device_id_type=pl.DeviceIdType.MESH,
            )

        for k in range(Q):
            @pl.when(my_z < Z - 1)
            def _():
                rdma(own(k), qslice(my_z, qi, k),
                     rsend.at[k], rrecv.at[k], z_right).start()

            @pl.when(my_z > 0)
            def _():
                rdma(own(k), qslice(my_z, qi, k),
                     lsend.at[k], lrecv.at[k], z_left).start()

        for t in range(1, Z):
            for k in range(Q):
                s = (t - 1) * Q + k

                @pl.when(t <= my_z)
                def _():
                    c = my_z - t
                    dst = qslice(c, qi, k)
                    rdma(dst, dst, rsend.at[0], rrecv.at[s], z_left).wait_recv()
                    rdma(dst, qslice(c, qi, k),
                         xdsend.at[s], xdrecv.at[s], x_dev).start()
                    rdma(dst, qslice(c, qi, k),
                         ydsend.at[s], ydrecv.at[s], y_dev).start()

                @pl.when(jnp.logical_and(t <= my_z, my_z < Z - 1))
                def _():
                    src = qslice(my_z - t, qi, k)
                    rdma(src, src, rsend.at[t * Q + k],
                         rrecv.at[t * Q + k], z_right).start()

                @pl.when(my_z + t <= Z - 1)
                def _():
                    c = my_z + t
                    dst = qslice(c, qi, k)
                    rdma(dst, dst, lsend.at[0], lrecv.at[s], z_right).wait_recv()
                    rdma(dst, qslice(c, qi, k),
                         xdsend.at[3 * Q + s], xdrecv.at[3 * Q + s],
                         x_dev).start()
                    rdma(dst, qslice(c, qi, k),
                         ydsend.at[3 * Q + s], ydrecv.at[3 * Q + s],
                         y_dev).start()

                @pl.when(jnp.logical_and(my_z + t <= Z - 1, my_z > 0))
                def _():
                    src = qslice(my_z + t, qi, k)
                    rdma(src, src, lsend.at[t * Q + k],
                         lrecv.at[t * Q + k], z_left).start()

        for t in range(1, Z):
            for k in range(Q):
                s = (t - 1) * Q + k
                for ss, cc, rightward in (
                    (s, my_z - t, True), (3 * Q + s, my_z + t, False)
                ):
                    guard = (t <= my_z) if rightward else (my_z + t <= Z - 1)

                    @pl.when(guard)
                    def _(ss=ss, cc=cc):
                        dx = qslice(cc, q_x, k)
                        rdma(dx, dx, xdsend.at[0], xdrecv.at[ss],
                             x_dev).wait_recv()
                        hsrc = hslice(cc, q_x, k, 0)
                        rdma(hsrc, hslice(cc, q_x, k, 0),
                             yhsend.at[ss], yhrecv.at[ss], y_dev).start()

                        dy = qslice(cc, q_y, k)
                        rdma(dy, dy, ydsend.at[0], ydrecv.at[ss],
                             y_dev).wait_recv()
                        hsrc2 = hslice(cc, q_y, k, 1)
                        rdma(hsrc2, hslice(cc, q_y, k, 1),
                             xhsend.at[ss], xhrecv.at[ss], x_dev).start()

        for t in range(1, Z):
            for k in range(Q):
                s = (t - 1) * Q + k
                for ss, cc, rightward in (
                    (s, my_z - t, True), (3 * Q + s, my_z + t, False)
                ):
                    guard = (t <= my_z) if rightward else (my_z + t <= Z - 1)

                    @pl.when(guard)
                    def _(ss=ss, cc=cc):
                        d0 = hslice(cc, q_d, k, 0)
                        rdma(d0, d0, yhsend.at[0], yhrecv.at[ss],
                             y_dev).wait_recv()
                        d1 = hslice(cc, q_d, k, 1)
                        rdma(d1, d1, xhsend.at[0], xhrecv.at[ss],
                             x_dev).wait_recv()

        for k in range(Q):
            @pl.when(my_z < Z - 1)
            def _():
                rdma(own(k), own(k), rsend.at[k], rrecv.at[0],
                     z_right).wait_send()

            @pl.when(my_z > 0)
            def _():
                rdma(own(k), own(k), lsend.at[k], lrecv.at[0],
                     z_left).wait_send()

        for t in range(1, Z):
            for k in range(Q):
                s = (t - 1) * Q + k

                @pl.when(jnp.logical_and(t <= my_z, my_z < Z - 1))
                def _():
                    src = qslice(my_z - t, qi, k)
                    rdma(src, src, rsend.at[t * Q + k], rrecv.at[0],
                         z_right).wait_send()

                @pl.when(jnp.logical_and(my_z + t <= Z - 1, my_z > 0))
                def _():
                    src = qslice(my_z + t, qi, k)
                    rdma(src, src, lsend.at[t * Q + k], lrecv.at[0],
                         z_left).wait_send()

                for ss, cc, rightward in (
                    (s, my_z - t, True), (3 * Q + s, my_z + t, False)
                ):
                    guard = (t <= my_z) if rightward else (my_z + t <= Z - 1)

                    @pl.when(guard)
                    def _(ss=ss, cc=cc):
                        src = qslice(cc, qi, k)
                        rdma(src, src, xdsend.at[ss], xdrecv.at[0],
                             x_dev).wait_send()
                        rdma(src, src, ydsend.at[ss], ydrecv.at[0],
                             y_dev).wait_send()
                        h0 = hslice(cc, q_x, k, 0)
                        rdma(h0, h0, yhsend.at[ss], yhrecv.at[0],
                             y_dev).wait_send()
                        h1 = hslice(cc, q_y, k, 1)
                        rdma(h1, h1, xhsend.at[ss], xhrecv.at[0],
                             x_dev).wait_send()

        local.wait()

    nsl = 3 * Q
    nxy = 6 * Q
    return pl.pallas_call(
        body,
        out_shape=jax.ShapeDtypeStruct((Z * m_per, n), x.dtype),
        in_specs=[pl.BlockSpec(memory_space=pltpu.MemorySpace.HBM)],
        out_specs=pl.BlockSpec(memory_space=pltpu.MemorySpace.HBM),
        scratch_shapes=[
            pltpu.SemaphoreType.DMA,
            pltpu.SemaphoreType.DMA((nsl,)),
            pltpu.SemaphoreType.DMA((nsl,)),
            pltpu.SemaphoreType.DMA((nsl,)),
            pltpu.SemaphoreType.DMA((nsl,)),
            pltpu.SemaphoreType.DMA((nxy,)),
            pltpu.SemaphoreType.DMA((nxy,)),
            pltpu.SemaphoreType.DMA((nxy,)),
            pltpu.SemaphoreType.DMA((nxy,)),
            pltpu.SemaphoreType.DMA((nxy,)),
            pltpu.SemaphoreType.DMA((nxy,)),
            pltpu.SemaphoreType.DMA((nxy,)),
            pltpu.SemaphoreType.DMA((nxy,)),
        ],
        compiler_params=pltpu.CompilerParams(collective_id=0),
    )(x)


# device time: 270973 ns/iter; 2.2352x vs baseline; 1.0120x over previous
import jax
import jax.numpy as jnp
from jax import lax
from jax.experimental import pallas as pl
from jax.experimental.pallas import tpu as pltpu

Z = 4
Q = 8


def kernel(x):
    m_per, n = x.shape
    qh = m_per // 4
    rp = qh // Q
    hp = rp // 2

    def body(x_ref, out_ref, copy_sem,
             rsend, rrecv, lsend, lrecv,
             xdsend, xdrecv, ydsend, ydrecv,
             xhsend, xhrecv, yhsend, yhrecv):
        my_x = lax.axis_index("x")
        my_y = lax.axis_index("y")
        my_z = lax.axis_index("z")
        qi = 2 * my_x + my_y
        q_x = 2 * (1 - my_x) + my_y
        q_y = 2 * my_x + (1 - my_y)
        q_d = 2 * (1 - my_x) + (1 - my_y)
        left = (my_z - 1) % Z
        right = (my_z + 1) % Z

        z_right = (my_x, my_y, my_z + 1)
        z_left = (my_x, my_y, my_z - 1)
        x_dev = (1 - my_x, my_y, my_z)
        y_dev = (my_x, 1 - my_y, my_z)

        local = pltpu.make_async_copy(
            x_ref, out_ref.at[pl.ds(my_z * m_per, m_per), :], copy_sem
        )
        local.start()

        barrier_sem = pltpu.get_barrier_semaphore()
        for nbr in ((my_x, my_y, left), (my_x, my_y, right), x_dev, y_dev):
            pl.semaphore_signal(
                barrier_sem, inc=1, device_id=nbr,
                device_id_type=pl.DeviceIdType.MESH,
            )
        pl.semaphore_wait(barrier_sem, 4)

        def qslice(c, q, k):
            return out_ref.at[pl.ds(c * m_per + q * qh + k * rp, rp), :]

        def hslice(c, q, k, hh):
            return out_ref.at[pl.ds(c * m_per + q * qh + k * rp + hh * hp, hp), :]

        def own(k):
            return x_ref.at[pl.ds(qi * qh + k * rp, rp), :]

        def rdma(src, dst, ssem, rsem, dev):
            return pltpu.make_async_remote_copy(
                src_ref=src, dst_ref=dst, send_sem=ssem, recv_sem=rsem,
                device_id=dev, device_id_type=pl.DeviceIdType.MESH,
            )

        for k in range(Q):
            @pl.when(my_z < Z - 1)
            def _():
                rdma(own(k), qslice(my_z, qi, k),
                     rsend.at[k], rrecv.at[k], z_right).start()

            @pl.when(my_z > 0)
            def _():
                rdma(own(k), qslice(my_z, qi, k),
                     lsend.at[k], lrecv.at[k], z_left).start()

        for t in range(1, Z):
            for k in range(Q):
                s = (t - 1) * Q + k

                @pl.when(t <= my_z)
                def _():
                    c = my_z - t
                    dst = qslice(c, qi, k)
                    rdma(dst, dst, rsend.at[0], rrecv.at[s], z_left).wait_recv()
                    rdma(dst, qslice(c, qi, k),
                         xdsend.at[s], xdrecv.at[s], x_dev).start()
                    rdma(dst, qslice(c, qi, k),
                         ydsend.at[s], ydrecv.at[s], y_dev).start()

                @pl.when(jnp.logical_and(t <= my_z, my_z < Z - 1))
                def _():
                    src = qslice(my_z - t, qi, k)
                    rdma(src, src, rsend.at[t * Q + k],
                         rrecv.at[t * Q + k], z_right).start()

                @pl.when(my_z + t <= Z - 1)
                def _():
                    c = my_z + t
                    dst = qslice(c, qi, k)
                    rdma(dst, dst, lsend.at[0], lrecv.at[s], z_right).wait_recv()
                    rdma(dst, qslice(c, qi, k),
                         xdsend.at[3 * Q + s], xdrecv.at[3 * Q + s],
                         x_dev).start()
                    rdma(dst, qslice(c, qi, k),
                         ydsend.at[3 * Q + s], ydrecv.at[3 * Q + s],
                         y_dev).start()

                @pl.when(jnp.logical_and(my_z + t <= Z - 1, my_z > 0))
                def _():
                    src = qslice(my_z + t, qi, k)
                    rdma(src, src, lsend.at[t * Q + k],
                         lrecv.at[t * Q + k], z_left).start()

        for t in range(1, Z):
            for k in range(Q):
                s = (t - 1) * Q + k
                for ss, cc, rightward in (
                    (s, my_z - t, True), (3 * Q + s, my_z + t, False)
                ):
                    guard = (t <= my_z) if rightward else (my_z + t <= Z - 1)

                    @pl.when(guard)
                    def _(ss=ss, cc=cc):
                        dx = qslice(cc, q_x, k)
                        rdma(dx, dx, xdsend.at[0], xdrecv.at[ss],
                             x_dev).wait_recv()
                        hsrc = hslice(cc, q_x, k, 0)
                        rdma(hsrc, hslice(cc, q_x, k, 0),
                             yhsend.at[ss], yhrecv.at[ss], y_dev).start()

                        dy = qslice(cc, q_y, k)
                        rdma(dy, dy, ydsend.at[0], ydrecv.at[ss],
                             y_dev).wait_recv()
                        hsrc2 = hslice(cc, q_y, k, 1)
                        rdma(hsrc2, hslice(cc, q_y, k, 1),
                             xhsend.at[ss], xhrecv.at[ss], x_dev).start()

        for t in range(1, Z):
            for k in range(Q):
                s = (t - 1) * Q + k
                for ss, cc, rightward in (
                    (s, my_z - t, True), (3 * Q + s, my_z + t, False)
                ):
                    guard = (t <= my_z) if rightward else (my_z + t <= Z - 1)

                    @pl.when(guard)
                    def _(ss=ss, cc=cc):
                        d0 = hslice(cc, q_d, k, 0)
                        rdma(d0, d0, yhsend.at[0], yhrecv.at[ss],
                             y_dev).wait_recv()
                        d1 = hslice(cc, q_d, k, 1)
                        rdma(d1, d1, xhsend.at[0], xhrecv.at[ss],
                             x_dev).wait_recv()

        for k in range(Q):
            @pl.when(my_z < Z - 1)
            def _():
                rdma(own(k), own(k), rsend.at[k], rrecv.at[0],
                     z_right).wait_send()

            @pl.when(my_z > 0)
            def _():
                rdma(own(k), own(k), lsend.at[k], lrecv.at[0],
                     z_left).wait_send()

        for t in range(1, Z):
            for k in range(Q):
                s = (t - 1) * Q + k

                @pl.when(jnp.logical_and(t <= my_z, my_z < Z - 1))
                def _():
                    src = qslice(my_z - t, qi, k)
                    rdma(src, src, rsend.at[t * Q + k], rrecv.at[0],
                         z_right).wait_send()

                @pl.when(jnp.logical_and(my_z + t <= Z - 1, my_z > 0))
                def _():
                    src = qslice(my_z + t, qi, k)
                    rdma(src, src, lsend.at[t * Q + k], lrecv.at[0],
                         z_left).wait_send()

                for ss, cc, rightward in (
                    (s, my_z - t, True), (3 * Q + s, my_z + t, False)
                ):
                    guard = (t <= my_z) if rightward else (my_z + t <= Z - 1)

                    @pl.when(guard)
                    def _(ss=ss, cc=cc):
                        src = qslice(cc, qi, k)
                        rdma(src, src, xdsend.at[ss], xdrecv.at[0],
                             x_dev).wait_send()
                        rdma(src, src, ydsend.at[ss], ydrecv.at[0],
                             y_dev).wait_send()
                        h0 = hslice(cc, q_x, k, 0)
                        rdma(h0, h0, yhsend.at[ss], yhrecv.at[0],
                             y_dev).wait_send()
                        h1 = hslice(cc, q_y, k, 1)
                        rdma(h1, h1, xhsend.at[ss], xhrecv.at[0],
                             x_dev).wait_send()

        local.wait()

    nsl = 3 * Q
    nxy = 6 * Q
    return pl.pallas_call(
        body,
        out_shape=jax.ShapeDtypeStruct((Z * m_per, n), x.dtype),
        in_specs=[pl.BlockSpec(memory_space=pltpu.MemorySpace.HBM)],
        out_specs=pl.BlockSpec(memory_space=pltpu.MemorySpace.HBM),
        scratch_shapes=[
            pltpu.SemaphoreType.DMA,
            pltpu.SemaphoreType.DMA((nsl,)),
            pltpu.SemaphoreType.DMA((nsl,)),
            pltpu.SemaphoreType.DMA((nsl,)),
            pltpu.SemaphoreType.DMA((nsl,)),
            pltpu.SemaphoreType.DMA((nxy,)),
            pltpu.SemaphoreType.DMA((nxy,)),
            pltpu.SemaphoreType.DMA((nxy,)),
            pltpu.SemaphoreType.DMA((nxy,)),
            pltpu.SemaphoreType.DMA((nxy,)),
            pltpu.SemaphoreType.DMA((nxy,)),
            pltpu.SemaphoreType.DMA((nxy,)),
            pltpu.SemaphoreType.DMA((nxy,)),
        ],
        compiler_params=pltpu.CompilerParams(collective_id=0),
    )(x)
